# baseline (speedup 1.0000x reference)
import functools

import jax
import jax.numpy as jnp
from jax import lax
from jax.experimental import pallas as pl
from jax.experimental.pallas import tpu as pltpu
from jax.experimental.pallas import tpu_sc as plsc

_NUM_CORES = 2
_NUM_SUBCORES = 16
_NUM_WORKERS = _NUM_CORES * _NUM_SUBCORES
_BB = 128
_NB = 4
_AHEAD = 2


@functools.partial(jax.jit, static_argnums=(2,))
def _sc_gather(idx, weight, n_chunks):
    mesh = plsc.VectorSubcoreMesh(core_axis_name="c", subcore_axis_name="s")

    @functools.partial(
        pl.kernel,
        mesh=mesh,
        out_type=jax.ShapeDtypeStruct((26, 8, 128, 8, 128), jnp.float32),
        scratch_types=[
            pltpu.VMEM((n_chunks, _BB), jnp.int32),
            pltpu.VMEM((_NB, _BB, 64), jnp.float32),
            pltpu.VMEM((_NB, 64, _BB), jnp.float32),
        ] + [pltpu.SemaphoreType.DMA] * (2 * _NB),
        compiler_params=pltpu.CompilerParams(
            use_tc_tiling_on_sc=False, needs_layout_passes=False
        ),
    )
    def k(idx_hbm, table_hbm, out_hbm, idx_v, rows_v, trans_v, *sems):
        gsems = sems[:_NB]
        wsems = sems[_NB:]
        wid = lax.axis_index("s") * _NUM_CORES + lax.axis_index("c")
        pltpu.sync_copy(idx_hbm.at[wid], idx_v)
        iota16 = lax.iota(jnp.int32, 16)

        for b in range(_NB):
            pltpu.async_copy(table_hbm.at[idx_v.at[b]], rows_v.at[b], gsems[b])

        def group(g, carry):
            j0 = g * _NB
            for b in range(_NB):
                j = j0 + b
                c_glob = wid * n_chunks + j
                f = c_glob // _BB
                b_blk = c_glob % _BB
                pltpu.make_async_copy(
                    table_hbm.at[idx_v.at[j]], rows_v.at[b], gsems[b]
                ).wait()

                def trow(c, carry2):
                    cidx = jnp.full((16,), c, jnp.int32)
                    for gg in range(_BB // 16):
                        v = plsc.load_gather(
                            rows_v.at[b], [gg * 16 + iota16, cidx]
                        )
                        trans_v[b, c, pl.ds(gg * 16, 16)] = v
                    return carry2

                lax.fori_loop(0, 64, trow, 0)

                for c0 in range(8):
                    pltpu.async_copy(
                        trans_v.at[b, pl.ds(c0 * 8, 8), :],
                        out_hbm.at[f, c0, b_blk],
                        wsems[b],
                    )

                jmid = j - _AHEAD
                bmid = (b - _AHEAD) % _NB

                @pl.when(jnp.logical_and(jmid >= 0, jmid + _NB < n_chunks))
                def _():
                    for c0 in range(8):
                        pltpu.make_async_copy(
                            trans_v.at[bmid, pl.ds(c0 * 8, 8), :],
                            out_hbm.at[0, 0, 0],
                            wsems[bmid],
                        ).wait()
                    pltpu.async_copy(
                        table_hbm.at[idx_v.at[jmid + _NB]],
                        rows_v.at[bmid],
                        gsems[bmid],
                    )

            return carry

        lax.fori_loop(0, n_chunks // _NB, group, 0)

        for b in range(_NB):
            for c0 in range(8):
                pltpu.make_async_copy(
                    trans_v.at[b, pl.ds(c0 * 8, 8), :],
                    out_hbm.at[0, 0, 0],
                    wsems[b],
                ).wait()

    return k(idx, weight)


def kernel(x, weight):
    b, f = x.shape
    bf = b * f
    assert bf % (_NUM_WORKERS * _BB * _NB) == 0 and weight.shape[1] == 64
    n_chunks = bf // (_NUM_WORKERS * _BB)
    idx = x.T.reshape(_NUM_WORKERS, n_chunks, _BB).astype(jnp.int32)
    out5 = _sc_gather(idx, weight, n_chunks)
    return out5.transpose(2, 4, 0, 1, 3).reshape(b, f, 64)


# speedup vs baseline: 1.1041x; 1.1041x over previous
import functools

import jax
import jax.numpy as jnp
from jax import lax
from jax.experimental import pallas as pl
from jax.experimental.pallas import tpu as pltpu
from jax.experimental.pallas import tpu_sc as plsc

_NUM_CORES = 2
_NUM_SUBCORES = 16
_NUM_WORKERS = _NUM_CORES * _NUM_SUBCORES
_BB = 128
_NB = 4
_AHEAD = 2


@functools.partial(jax.jit, static_argnums=(2,))
def _sc_gather(idx, weight, n_chunks):
    mesh = plsc.VectorSubcoreMesh(core_axis_name="c", subcore_axis_name="s")

    @functools.partial(
        pl.kernel,
        mesh=mesh,
        out_type=jax.ShapeDtypeStruct((26, 8, 128, 1024), jnp.float32),
        scratch_types=[
            pltpu.VMEM((n_chunks, _BB), jnp.int32),
            pltpu.VMEM((_NB, _BB, 64), jnp.float32),
            pltpu.VMEM((_NB, 64 * _BB), jnp.float32),
        ] + [pltpu.SemaphoreType.DMA] * (2 * _NB),
        compiler_params=pltpu.CompilerParams(
            use_tc_tiling_on_sc=False, needs_layout_passes=False
        ),
    )
    def k(idx_hbm, table_hbm, out_hbm, idx_v, rows_v, trans_v, *sems):
        gsems = sems[:_NB]
        wsems = sems[_NB:]
        wid = lax.axis_index("s") * _NUM_CORES + lax.axis_index("c")
        pltpu.sync_copy(idx_hbm.at[wid], idx_v)
        iota128 = lax.iota(jnp.int32, 16) * _BB

        for b in range(_NB):
            pltpu.async_copy(table_hbm.at[idx_v.at[b]], rows_v.at[b], gsems[b])

        def group(g, carry):
            j0 = g * _NB
            for b in range(_NB):
                j = j0 + b
                c_glob = wid * n_chunks + j
                f = c_glob // _BB
                b_blk = c_glob % _BB
                pltpu.make_async_copy(
                    table_hbm.at[idx_v.at[j]], rows_v.at[b], gsems[b]
                ).wait()

                def trow(bb4, carry2):
                    for u in range(4):
                        bb = bb4 * 4 + u
                        for cg in range(4):
                            v = rows_v[b, bb, pl.ds(cg * 16, 16)]
                            plsc.store_scatter(
                                trans_v.at[b],
                                [bb + cg * 16 * _BB + iota128],
                                v,
                            )
                    return carry2

                lax.fori_loop(0, _BB // 4, trow, 0)

                for c0 in range(8):
                    pltpu.async_copy(
                        trans_v.at[b, pl.ds(c0 * 1024, 1024)],
                        out_hbm.at[f, c0, b_blk],
                        wsems[b],
                    )

                jmid = j - _AHEAD
                bmid = (b - _AHEAD) % _NB

                @pl.when(jnp.logical_and(jmid >= 0, jmid + _NB < n_chunks))
                def _():
                    for c0 in range(8):
                        pltpu.make_async_copy(
                            trans_v.at[bmid, pl.ds(c0 * 1024, 1024)],
                            out_hbm.at[0, 0, 0],
                            wsems[bmid],
                        ).wait()
                    pltpu.async_copy(
                        table_hbm.at[idx_v.at[jmid + _NB]],
                        rows_v.at[bmid],
                        gsems[bmid],
                    )

            return carry

        lax.fori_loop(0, n_chunks // _NB, group, 0)

        for b in range(_NB):
            for c0 in range(8):
                pltpu.make_async_copy(
                    trans_v.at[b, pl.ds(c0 * 1024, 1024)],
                    out_hbm.at[0, 0, 0],
                    wsems[b],
                ).wait()

    return k(idx, weight)


def kernel(x, weight):
    b, f = x.shape
    bf = b * f
    assert bf % (_NUM_WORKERS * _BB * _NB) == 0 and weight.shape[1] == 64
    n_chunks = bf // (_NUM_WORKERS * _BB)
    idx = x.T.reshape(_NUM_WORKERS, n_chunks, _BB).astype(jnp.int32)
    out4 = _sc_gather(idx, weight, n_chunks)
    return (
        out4.reshape(26, 8, 128, 8, 128)
        .transpose(2, 4, 0, 1, 3)
        .reshape(b, f, 64)
    )


# speedup vs baseline: 1.1736x; 1.0629x over previous
import functools

import jax
import jax.numpy as jnp
from jax import lax
from jax.experimental import pallas as pl
from jax.experimental.pallas import tpu as pltpu
from jax.experimental.pallas import tpu_sc as plsc

_NUM_CORES = 2
_NUM_SUBCORES = 16
_NUM_WORKERS = _NUM_CORES * _NUM_SUBCORES
_CHUNK = 128
_NB = 8
_AHEAD = 4

_BP = 512


def _tc_transpose(wt, n_out):
    def body(in_ref, out_ref):
        xt = jnp.swapaxes(in_ref[...], 0, 1)
        out_ref[...] = jnp.concatenate([xt[0:_BP], xt[_BP:2 * _BP]], axis=1)

    return pl.pallas_call(
        body,
        grid=(n_out // _BP,),
        in_specs=[pl.BlockSpec((64, 2 * _BP), lambda i: (0, i))],
        out_specs=pl.BlockSpec((_BP, 128), lambda i: (i, 0)),
        out_shape=jax.ShapeDtypeStruct((n_out, 128), jnp.float32),
    )(wt)


@functools.partial(jax.jit, static_argnums=(2, 3))
def _sc_gather(idx, table, n_chunks, d):
    mesh = plsc.VectorSubcoreMesh(core_axis_name="c", subcore_axis_name="s")

    @functools.partial(
        pl.kernel,
        mesh=mesh,
        out_type=jax.ShapeDtypeStruct(
            (_NUM_WORKERS * n_chunks, _CHUNK, d), jnp.float32
        ),
        scratch_types=[
            pltpu.VMEM((n_chunks, _CHUNK), jnp.int32),
            pltpu.VMEM((_NB, _CHUNK, d), jnp.float32),
        ] + [pltpu.SemaphoreType.DMA] * (2 * _NB),
        compiler_params=pltpu.CompilerParams(use_tc_tiling_on_sc=False),
    )
    def k(idx_hbm, table_hbm, out_hbm, idx_v, rows_v, *sems):
        gsems = sems[:_NB]
        wsems = sems[_NB:]
        wid = lax.axis_index("s") * _NUM_CORES + lax.axis_index("c")
        base = wid * n_chunks
        pltpu.sync_copy(idx_hbm.at[wid], idx_v)

        for b in range(_NB):
            pltpu.async_copy(table_hbm.at[idx_v.at[b]], rows_v.at[b], gsems[b])

        def group(g, carry):
            j0 = g * _NB
            for b in range(_NB):
                j = j0 + b
                pltpu.make_async_copy(
                    table_hbm.at[idx_v.at[j]], rows_v.at[b], gsems[b]
                ).wait()
                pltpu.async_copy(rows_v.at[b], out_hbm.at[base + j], wsems[b])
                jmid = j - _AHEAD
                bmid = (b - _AHEAD) % _NB

                @pl.when(jnp.logical_and(jmid >= 0, jmid + _NB < n_chunks))
                def _():
                    pltpu.make_async_copy(
                        rows_v.at[bmid], out_hbm.at[base], wsems[bmid]
                    ).wait()
                    pltpu.async_copy(
                        table_hbm.at[idx_v.at[jmid + _NB]],
                        rows_v.at[bmid],
                        gsems[bmid],
                    )

            return carry

        lax.fori_loop(0, n_chunks // _NB, group, 0)

        for b in range(_NB):
            pltpu.make_async_copy(
                rows_v.at[b], out_hbm.at[base], wsems[b]
            ).wait()

    return k(idx, table)


def kernel(x, weight):
    b, f = x.shape
    v, d = weight.shape
    bf = b * f
    assert bf % (_NUM_WORKERS * _CHUNK * _NB) == 0 and d == 64
    n_chunks = bf // (_NUM_WORKERS * _CHUNK)

    n_out = -(-v // (2 * _BP)) * _BP
    table = _tc_transpose(weight.T, n_out).reshape(2 * n_out, d)

    r = x.astype(jnp.int32)
    i2 = (r // (2 * _BP)) * (2 * _BP)
    rr = r - i2
    m = jnp.where(rr < _BP, i2 + 2 * rr, i2 + 2 * (rr - _BP) + 1)
    idx = m.reshape(_NUM_WORKERS, n_chunks, _CHUNK)

    out = _sc_gather(idx, table, n_chunks, d)
    return out.reshape(b, f, d)
